# initial kernel scaffold (unmeasured)
import jax
import jax.numpy as jnp
from jax import lax
from jax.experimental import pallas as pl
from jax.experimental.pallas import tpu as pltpu

N_DEV = 4
N_HOPS = 2 * (N_DEV - 1)


def _gelu(y):
    c = 0.7978845608028654
    return 0.5 * y * (1.0 + jnp.tanh(c * (y + 0.044715 * y ** 3)))


def kernel(x, w_mat):
    m, k_per = x.shape
    _, n = w_mat.shape
    chunk = m // N_DEV

    def body(x_ref, w_ref, out_ref, acc_ref, comm_ref, send_sems, recv_sems):
        my = lax.axis_index("i")
        left = lax.rem(my + N_DEV - 1, N_DEV)
        right = lax.rem(my + 1, N_DEV)

        barrier_sem = pltpu.get_barrier_semaphore()
        for nbr in (left, right):
            pl.semaphore_signal(
                barrier_sem, inc=1,
                device_id=(nbr,), device_id_type=pl.DeviceIdType.MESH,
            )
        pl.semaphore_wait(barrier_sem, 2)

        acc_ref[...] = jnp.dot(
            x_ref[...], w_ref[...], preferred_element_type=jnp.float32
        )

        def chunk_off(idx):
            return lax.rem(idx + 4 * N_DEV, N_DEV) * chunk

        comm_ref[0] = acc_ref[pl.ds(chunk_off(my), chunk), :].astype(jnp.bfloat16)

        for h in range(N_HOPS):
            rdma = pltpu.make_async_remote_copy(
                src_ref=comm_ref.at[h],
                dst_ref=comm_ref.at[h + 1],
                send_sem=send_sems.at[h],
                recv_sem=recv_sems.at[h],
                device_id=(right,),
                device_id_type=pl.DeviceIdType.MESH,
            )
            rdma.start()
            rdma.wait()

            if h < N_DEV - 1:
                off = chunk_off(my - h - 1)
                summed = comm_ref[h + 1].astype(jnp.float32) + acc_ref[
                    pl.ds(off, chunk), :
                ]
                if h < N_DEV - 2:
                    comm_ref[h + 1] = summed.astype(jnp.bfloat16)
                else:
                    g = _gelu(summed)
                    out_ref[pl.ds(chunk_off(my + 1), chunk), :] = g
                    comm_ref[h + 1] = g.astype(jnp.bfloat16)
            else:
                off = chunk_off(my - (h - (N_DEV - 1)))
                out_ref[pl.ds(off, chunk), :] = comm_ref[h + 1].astype(jnp.float32)

    return pl.pallas_call(
        body,
        out_shape=jax.ShapeDtypeStruct((m, n), jnp.float32),
        in_specs=[
            pl.BlockSpec(memory_space=pltpu.VMEM),
            pl.BlockSpec(memory_space=pltpu.VMEM),
        ],
        out_specs=pl.BlockSpec(memory_space=pltpu.VMEM),
        scratch_shapes=[
            pltpu.VMEM((m, n), jnp.float32),
            pltpu.VMEM((N_HOPS + 1, chunk, n), jnp.bfloat16),
            pltpu.SemaphoreType.DMA((N_HOPS,)),
            pltpu.SemaphoreType.DMA((N_HOPS,)),
        ],
        compiler_params=pltpu.CompilerParams(collective_id=0),
    )(x, w_mat)


# baseline (device time: 174992 ns/iter reference)
import jax
import jax.numpy as jnp
from jax import lax
from jax.experimental import pallas as pl
from jax.experimental.pallas import tpu as pltpu

N_DEV = 4
N_HOPS = 2 * (N_DEV - 1)


def _gelu(y):
    c = 0.7978845608028654
    return 0.5 * y * (1.0 + jnp.tanh(c * (y + 0.044715 * y ** 3)))


def kernel(x, w_mat):
    m, k_per = x.shape
    _, n = w_mat.shape
    chunk = m // N_DEV

    def body(x_ref, w_ref, out_ref, wbf_ref, comm_ref, send_sems, recv_sems):
        my = lax.axis_index("i")
        left = lax.rem(my + N_DEV - 1, N_DEV)
        right = lax.rem(my + 1, N_DEV)

        barrier_sem = pltpu.get_barrier_semaphore()
        for nbr in (left, right):
            pl.semaphore_signal(
                barrier_sem, inc=1,
                device_id=(nbr,), device_id_type=pl.DeviceIdType.MESH,
            )
        pl.semaphore_wait(barrier_sem, 2)

        wbf_ref[...] = w_ref[...].astype(jnp.bfloat16)

        def chunk_off(idx):
            return lax.rem(idx + 4 * N_DEV, N_DEV) * chunk

        def pchunk(idx):
            off = chunk_off(idx)
            return jnp.dot(
                x_ref[pl.ds(off, chunk), :].astype(jnp.bfloat16),
                wbf_ref[...],
                preferred_element_type=jnp.float32,
            )

        comm_ref[0] = pchunk(my).astype(jnp.bfloat16)

        for h in range(N_HOPS):
            rdma = pltpu.make_async_remote_copy(
                src_ref=comm_ref.at[h],
                dst_ref=comm_ref.at[h + 1],
                send_sem=send_sems.at[h],
                recv_sem=recv_sems.at[h],
                device_id=(right,),
                device_id_type=pl.DeviceIdType.MESH,
            )
            rdma.start()
            if h < N_DEV - 1:
                p = pchunk(my - h - 1)
            rdma.wait()

            if h < N_DEV - 1:
                summed = comm_ref[h + 1].astype(jnp.float32) + p
                if h < N_DEV - 2:
                    comm_ref[h + 1] = summed.astype(jnp.bfloat16)
                else:
                    g = _gelu(summed)
                    out_ref[pl.ds(chunk_off(my + 1), chunk), :] = g
                    comm_ref[h + 1] = g.astype(jnp.bfloat16)
            else:
                off = chunk_off(my - (h - (N_DEV - 1)))
                out_ref[pl.ds(off, chunk), :] = comm_ref[h + 1].astype(jnp.float32)

    return pl.pallas_call(
        body,
        out_shape=jax.ShapeDtypeStruct((m, n), jnp.float32),
        in_specs=[
            pl.BlockSpec(memory_space=pltpu.VMEM),
            pl.BlockSpec(memory_space=pltpu.VMEM),
        ],
        out_specs=pl.BlockSpec(memory_space=pltpu.VMEM),
        scratch_shapes=[
            pltpu.VMEM((k_per, n), jnp.bfloat16),
            pltpu.VMEM((N_HOPS + 1, chunk, n), jnp.bfloat16),
            pltpu.SemaphoreType.DMA((N_HOPS,)),
            pltpu.SemaphoreType.DMA((N_HOPS,)),
        ],
        compiler_params=pltpu.CompilerParams(
            collective_id=0,
            vmem_limit_bytes=100 * 1024 * 1024,
        ),
    )(x, w_mat)


# device time: 107088 ns/iter; 1.6341x vs baseline; 1.6341x over previous
import jax
import jax.numpy as jnp
from jax import lax
from jax.experimental import pallas as pl
from jax.experimental.pallas import tpu as pltpu

N_DEV = 4
N_HOPS = 2 * (N_DEV - 1)


def _gelu(y):
    c = 0.7978845608028654
    return 0.5 * y * (1.0 + jnp.tanh(c * (y + 0.044715 * y ** 3)))


def kernel(x, w_mat):
    m, k_per = x.shape
    _, n = w_mat.shape
    chunk = m // N_DEV
    n2 = n // 2

    def body(x_ref, w_ref, out_ref, wbf_ref,
             comm_cw, comm_ccw, ssem_cw, rsem_cw, ssem_ccw, rsem_ccw):
        my = lax.axis_index("i")
        left = lax.rem(my + N_DEV - 1, N_DEV)
        right = lax.rem(my + 1, N_DEV)

        barrier_sem = pltpu.get_barrier_semaphore()
        for nbr in (left, right):
            pl.semaphore_signal(
                barrier_sem, inc=1,
                device_id=(nbr,), device_id_type=pl.DeviceIdType.MESH,
            )
        pl.semaphore_wait(barrier_sem, 2)

        wbf_ref[...] = w_ref[...].astype(jnp.bfloat16)

        def chunk_off(idx):
            return lax.rem(idx + 4 * N_DEV, N_DEV) * chunk

        def pchunk(idx, c0):
            off = chunk_off(idx)
            return jnp.dot(
                x_ref[pl.ds(off, chunk), :].astype(jnp.bfloat16),
                wbf_ref[:, c0:c0 + n2],
                preferred_element_type=jnp.float32,
            )

        dirs = [
            dict(comm=comm_cw, ssem=ssem_cw, rsem=rsem_cw,
                 tgt=right, sign=-1, c0=0),
            dict(comm=comm_ccw, ssem=ssem_ccw, rsem=rsem_ccw,
                 tgt=left, sign=1, c0=n2),
        ]
        descs = [
            [
                pltpu.make_async_remote_copy(
                    src_ref=d["comm"].at[h],
                    dst_ref=d["comm"].at[h + 1],
                    send_sem=d["ssem"].at[h],
                    recv_sem=d["rsem"].at[h],
                    device_id=(d["tgt"],),
                    device_id_type=pl.DeviceIdType.MESH,
                )
                for h in range(N_HOPS)
            ]
            for d in dirs
        ]

        for d in dirs:
            d["comm"][0] = pchunk(my, d["c0"]).astype(jnp.bfloat16)
        for dd in descs:
            dd[0].start()

        for h in range(N_DEV - 1):
            pvals = [pchunk(my + d["sign"] * (h + 1), d["c0"]) for d in dirs]
            for dd in descs:
                dd[h].wait()
            for d, pv in zip(dirs, pvals):
                summed = d["comm"][h + 1].astype(jnp.float32) + pv
                if h < N_DEV - 2:
                    d["comm"][h + 1] = summed.astype(jnp.bfloat16)
                else:
                    g = _gelu(summed)
                    off = chunk_off(my - d["sign"])
                    out_ref[pl.ds(off, chunk), d["c0"]:d["c0"] + n2] = g
                    d["comm"][h + 1] = g.astype(jnp.bfloat16)
            for dd in descs:
                dd[h + 1].start()

        for h in range(N_DEV - 1, N_HOPS):
            for dd in descs:
                dd[h].wait()
            if h + 1 < N_HOPS:
                for dd in descs:
                    dd[h + 1].start()
            for d in dirs:
                off = chunk_off(my + d["sign"] * (h - (N_DEV - 1)))
                out_ref[pl.ds(off, chunk), d["c0"]:d["c0"] + n2] = (
                    d["comm"][h + 1].astype(jnp.float32)
                )

    return pl.pallas_call(
        body,
        out_shape=jax.ShapeDtypeStruct((m, n), jnp.float32),
        in_specs=[
            pl.BlockSpec(memory_space=pltpu.VMEM),
            pl.BlockSpec(memory_space=pltpu.VMEM),
        ],
        out_specs=pl.BlockSpec(memory_space=pltpu.VMEM),
        scratch_shapes=[
            pltpu.VMEM((k_per, n), jnp.bfloat16),
            pltpu.VMEM((N_HOPS + 1, chunk, n2), jnp.bfloat16),
            pltpu.VMEM((N_HOPS + 1, chunk, n2), jnp.bfloat16),
            pltpu.SemaphoreType.DMA((N_HOPS,)),
            pltpu.SemaphoreType.DMA((N_HOPS,)),
            pltpu.SemaphoreType.DMA((N_HOPS,)),
            pltpu.SemaphoreType.DMA((N_HOPS,)),
        ],
        compiler_params=pltpu.CompilerParams(
            collective_id=0,
            vmem_limit_bytes=100 * 1024 * 1024,
        ),
    )(x, w_mat)


# device time: 95558 ns/iter; 1.8313x vs baseline; 1.1207x over previous
import jax
import jax.numpy as jnp
from jax import lax
from jax.experimental import pallas as pl
from jax.experimental.pallas import tpu as pltpu

N_DEV = 4
N_HOPS = 2 * (N_DEV - 1)
N_SUB = 2


def _gelu(y):
    c = 0.7978845608028654
    return 0.5 * y * (1.0 + jnp.tanh(c * (y + 0.044715 * y ** 3)))


def kernel(x, w_mat):
    m, k_per = x.shape
    _, n = w_mat.shape
    chunk = m // N_DEV
    n2 = n // 2
    sc = n2 // N_SUB

    def body(x_ref, w_ref, out_ref, wbf_ref,
             comm_cw, comm_ccw, ssem_cw, rsem_cw, ssem_ccw, rsem_ccw):
        my = lax.axis_index("i")
        left = lax.rem(my + N_DEV - 1, N_DEV)
        right = lax.rem(my + 1, N_DEV)

        barrier_sem = pltpu.get_barrier_semaphore()
        for nbr in (left, right):
            pl.semaphore_signal(
                barrier_sem, inc=1,
                device_id=(nbr,), device_id_type=pl.DeviceIdType.MESH,
            )
        pl.semaphore_wait(barrier_sem, 2)

        wbf_ref[...] = w_ref[...].astype(jnp.bfloat16)

        def chunk_off(idx):
            return lax.rem(idx + 4 * N_DEV, N_DEV) * chunk

        dirs = [
            dict(comm=comm_cw, ssem=ssem_cw, rsem=rsem_cw,
                 tgt=right, sign=-1, c0=0),
            dict(comm=comm_ccw, ssem=ssem_ccw, rsem=rsem_ccw,
                 tgt=left, sign=1, c0=n2),
        ]
        descs = [
            [
                [
                    pltpu.make_async_remote_copy(
                        src_ref=d["comm"].at[h, s],
                        dst_ref=d["comm"].at[h + 1, s],
                        send_sem=d["ssem"].at[h, s],
                        recv_sem=d["rsem"].at[h, s],
                        device_id=(d["tgt"],),
                        device_id_type=pl.DeviceIdType.MESH,
                    )
                    for s in range(N_SUB)
                ]
                for h in range(N_HOPS)
            ]
            for d in dirs
        ]

        def xrows(idx):
            return x_ref[pl.ds(chunk_off(idx), chunk), :].astype(jnp.bfloat16)

        def wcols(d, s):
            c = d["c0"] + s * sc
            return wbf_ref[:, c:c + sc]

        xb0 = xrows(my)
        for s in range(N_SUB):
            for i, d in enumerate(dirs):
                d["comm"][0, s] = jnp.dot(
                    xb0, wcols(d, s), preferred_element_type=jnp.float32
                ).astype(jnp.bfloat16)
            for i in range(2):
                descs[i][0][s].start()

        for h in range(N_DEV - 1):
            xbs = [xrows(my + d["sign"] * (h + 1)) for d in dirs]
            ps = [
                [
                    jnp.dot(xbs[i], wcols(d, s),
                            preferred_element_type=jnp.float32)
                    for s in range(N_SUB)
                ]
                for i, d in enumerate(dirs)
            ]
            for s in range(N_SUB):
                for i in range(2):
                    descs[i][h][s].wait()
                for i, d in enumerate(dirs):
                    summed = d["comm"][h + 1, s].astype(jnp.float32) + ps[i][s]
                    if h < N_DEV - 2:
                        d["comm"][h + 1, s] = summed.astype(jnp.bfloat16)
                    else:
                        g = _gelu(summed)
                        off = chunk_off(my - d["sign"])
                        c = d["c0"] + s * sc
                        out_ref[pl.ds(off, chunk), c:c + sc] = g
                        d["comm"][h + 1, s] = g.astype(jnp.bfloat16)
                for i in range(2):
                    descs[i][h + 1][s].start()

        for h in range(N_DEV - 1, N_HOPS):
            for s in range(N_SUB):
                for i in range(2):
                    descs[i][h][s].wait()
                if h + 1 < N_HOPS:
                    for i in range(2):
                        descs[i][h + 1][s].start()
                for i, d in enumerate(dirs):
                    off = chunk_off(my + d["sign"] * (h - (N_DEV - 1)))
                    c = d["c0"] + s * sc
                    out_ref[pl.ds(off, chunk), c:c + sc] = (
                        d["comm"][h + 1, s].astype(jnp.float32)
                    )

    return pl.pallas_call(
        body,
        out_shape=jax.ShapeDtypeStruct((m, n), jnp.float32),
        in_specs=[
            pl.BlockSpec(memory_space=pltpu.VMEM),
            pl.BlockSpec(memory_space=pltpu.VMEM),
        ],
        out_specs=pl.BlockSpec(memory_space=pltpu.VMEM),
        scratch_shapes=[
            pltpu.VMEM((k_per, n), jnp.bfloat16),
            pltpu.VMEM((N_HOPS + 1, N_SUB, chunk, sc), jnp.bfloat16),
            pltpu.VMEM((N_HOPS + 1, N_SUB, chunk, sc), jnp.bfloat16),
            pltpu.SemaphoreType.DMA((N_HOPS, N_SUB)),
            pltpu.SemaphoreType.DMA((N_HOPS, N_SUB)),
            pltpu.SemaphoreType.DMA((N_HOPS, N_SUB)),
            pltpu.SemaphoreType.DMA((N_HOPS, N_SUB)),
        ],
        compiler_params=pltpu.CompilerParams(
            collective_id=0,
            vmem_limit_bytes=100 * 1024 * 1024,
        ),
    )(x, w_mat)


# device time: 87628 ns/iter; 1.9970x vs baseline; 1.0905x over previous
import jax
import jax.numpy as jnp
from jax import lax
from jax.experimental import pallas as pl
from jax.experimental.pallas import tpu as pltpu

N_DEV = 4
N_RS = N_DEV - 1
N_HOPS = 2 * (N_DEV - 1)
N_SUB = 2


def _gelu(y):
    c = 0.7978845608028654
    return 0.5 * y * (1.0 + jnp.tanh(c * (y + 0.044715 * y ** 3)))


def kernel(x, w_mat):
    m, k_per = x.shape
    _, n = w_mat.shape
    chunk = m // N_DEV
    n2 = n // 2
    sc = n2 // N_SUB

    def body(x_ref, w_ref, out_ref, wbf_ref,
             comm_cw, comm_ccw, ssem_cw, rsem_cw, ssem_ccw, rsem_ccw):
        my = lax.axis_index("i")
        left = lax.rem(my + N_DEV - 1, N_DEV)
        right = lax.rem(my + 1, N_DEV)

        barrier_sem = pltpu.get_barrier_semaphore()
        for nbr in (left, right):
            pl.semaphore_signal(
                barrier_sem, inc=1,
                device_id=(nbr,), device_id_type=pl.DeviceIdType.MESH,
            )
        pl.semaphore_wait(barrier_sem, 2)

        wbf_ref[...] = w_ref[...].astype(jnp.bfloat16)

        def chunk_off(idx):
            return lax.rem(idx + 4 * N_DEV, N_DEV) * chunk

        dirs = [
            dict(comm=comm_cw, ssem=ssem_cw, rsem=rsem_cw,
                 tgt=right, sign=-1, c0=0),
            dict(comm=comm_ccw, ssem=ssem_ccw, rsem=rsem_ccw,
                 tgt=left, sign=1, c0=n2),
        ]

        def out_slice(d, idx, s):
            c = d["c0"] + s * sc
            return out_ref.at[pl.ds(chunk_off(idx), chunk), pl.ds(c, sc)]

        def make_desc(d, h, s):
            if h < N_RS:
                src = d["comm"].at[h, s]
                dst = d["comm"].at[h + 1, s]
            else:
                src = out_slice(d, my + d["sign"] * (h - 4), s)
                dst = out_slice(d, my + d["sign"] * (h - 4), s)
            return pltpu.make_async_remote_copy(
                src_ref=src, dst_ref=dst,
                send_sem=d["ssem"].at[h, s],
                recv_sem=d["rsem"].at[h, s],
                device_id=(d["tgt"],),
                device_id_type=pl.DeviceIdType.MESH,
            )

        descs = [
            [[make_desc(d, h, s) for s in range(N_SUB)] for h in range(N_HOPS)]
            for d in dirs
        ]
        recv_descs = [
            [
                [
                    pltpu.make_async_remote_copy(
                        src_ref=out_slice(d, my + d["sign"] * (h - 3), s),
                        dst_ref=out_slice(d, my + d["sign"] * (h - 3), s),
                        send_sem=d["ssem"].at[h, s],
                        recv_sem=d["rsem"].at[h, s],
                        device_id=(d["tgt"],),
                        device_id_type=pl.DeviceIdType.MESH,
                    )
                    for s in range(N_SUB)
                ]
                for h in range(N_RS, N_HOPS)
            ]
            for d in dirs
        ]

        def xrows(idx):
            return x_ref[pl.ds(chunk_off(idx), chunk), :].astype(jnp.bfloat16)

        def wcols(d, s):
            c = d["c0"] + s * sc
            return wbf_ref[:, c:c + sc]

        xb0 = xrows(my)
        for s in range(N_SUB):
            for d in dirs:
                d["comm"][0, s] = jnp.dot(
                    xb0, wcols(d, s), preferred_element_type=jnp.float32
                ).astype(jnp.bfloat16)
            for i in range(2):
                descs[i][0][s].start()

        for h in range(N_RS):
            xbs = [xrows(my + d["sign"] * (h + 1)) for d in dirs]
            ps = [
                [
                    jnp.dot(xbs[i], wcols(d, s),
                            preferred_element_type=jnp.float32)
                    for s in range(N_SUB)
                ]
                for i, d in enumerate(dirs)
            ]
            for s in range(N_SUB):
                for i in range(2):
                    descs[i][h][s].wait()
                for i, d in enumerate(dirs):
                    summed = d["comm"][h + 1, s].astype(jnp.float32) + ps[i][s]
                    if h < N_RS - 1:
                        d["comm"][h + 1, s] = summed.astype(jnp.bfloat16)
                    else:
                        g = _gelu(summed).astype(jnp.bfloat16)
                        off = chunk_off(my - d["sign"])
                        c = d["c0"] + s * sc
                        out_ref[pl.ds(off, chunk), c:c + sc] = g
                for i in range(2):
                    descs[i][h + 1][s].start()

        for h in range(N_RS, N_HOPS):
            for s in range(N_SUB):
                for i in range(2):
                    recv_descs[i][h - N_RS][s].wait_recv()
                if h + 1 < N_HOPS:
                    for i in range(2):
                        descs[i][h + 1][s].start()

        for h in range(N_RS, N_HOPS):
            for s in range(N_SUB):
                for i in range(2):
                    descs[i][h][s].wait_send()

    return pl.pallas_call(
        body,
        out_shape=jax.ShapeDtypeStruct((m, n), jnp.bfloat16),
        in_specs=[
            pl.BlockSpec(memory_space=pltpu.VMEM),
            pl.BlockSpec(memory_space=pltpu.VMEM),
        ],
        out_specs=pl.BlockSpec(memory_space=pltpu.VMEM),
        scratch_shapes=[
            pltpu.VMEM((k_per, n), jnp.bfloat16),
            pltpu.VMEM((N_RS + 1, N_SUB, chunk, sc), jnp.bfloat16),
            pltpu.VMEM((N_RS + 1, N_SUB, chunk, sc), jnp.bfloat16),
            pltpu.SemaphoreType.DMA((N_HOPS, N_SUB)),
            pltpu.SemaphoreType.DMA((N_HOPS, N_SUB)),
            pltpu.SemaphoreType.DMA((N_HOPS, N_SUB)),
            pltpu.SemaphoreType.DMA((N_HOPS, N_SUB)),
        ],
        compiler_params=pltpu.CompilerParams(
            collective_id=0,
            vmem_limit_bytes=100 * 1024 * 1024,
        ),
    )(x, w_mat)
